# baseline (speedup 1.0000x reference)
import functools
import jax
import jax.numpy as jnp
from jax import lax
from jax.experimental import pallas as pl
from jax.experimental.pallas import tpu as pltpu, tpu_sc as plsc

_B, _S, _H = 4, 4096, 1024
_S1 = 2048
_NC, _NS = 2, 16
_NW = _NC * _NS
_ROWS_W = _S1 // _NW
_R = 8
_NCH = _ROWS_W // _R
_TC_BS = 1024


def _sc_body(x_hbm, pos_hbm, out_hbm,
             pos0, pos1, xa0, xb0, xc0, xd0, xa1, xb1, xc1, xd1,
             sin0, sin1, sout0, sout1):
    wid = lax.axis_index("c") * _NS + lax.axis_index("s")
    posv = (pos0, pos1)
    xv = ((xa0, xb0, xc0, xd0), (xa1, xb1, xc1, xd1))
    sin = (sin0, sin1)
    sout = (sout0, sout1)

    def rbase(g):
        return wid * _ROWS_W + g * _R

    def issue_in(g, p):
        pltpu.async_copy(pos_hbm.at[pl.ds(rbase(g), _R), :], posv[p], sin[p])
        for b in range(_B):
            pltpu.async_copy(x_hbm.at[b, pl.ds(rbase(g), _R), :], xv[p][b], sin[p])

    def drain_in(p):
        pltpu.make_async_copy(pos_hbm.at[pl.ds(0, _R), :], posv[p], sin[p]).wait()
        for b in range(_B):
            pltpu.make_async_copy(x_hbm.at[0, pl.ds(0, _R), :], xv[p][b], sin[p]).wait()

    def issue_out(g, p):
        for b in range(_B):
            pltpu.async_copy(xv[p][b], out_hbm.at[b, pl.ds(rbase(g), _R), :], sout[p])

    def drain_out(p):
        for b in range(_B):
            pltpu.make_async_copy(xv[p][b], out_hbm.at[0, pl.ds(0, _R), :], sout[p]).wait()

    def compute(p):
        bufs = xv[p]
        pv_ref = posv[p]

        def row_body(r, acc):
            def col_body(j, acc2):
                cs = j * 16
                pv = pv_ref[r, pl.ds(cs, 16)]
                for b in range(_B):
                    plsc.addupdate(bufs[b].at[r, pl.ds(cs, 16)], pv)
                return acc2

            return lax.fori_loop(0, _H // 16, col_body, acc, unroll=8)

        lax.fori_loop(0, _R, row_body, 0)

    issue_in(0, 0)
    for g in range(_NCH):
        p = g % 2
        if g + 1 < _NCH:
            if g >= 1:
                drain_out(1 - p)
            issue_in(g + 1, 1 - p)
        drain_in(p)
        compute(p)
        issue_out(g, p)
    drain_out(0)
    drain_out(1)


def _tc_body(x_ref, p_ref, o_ref):
    o_ref[...] = x_ref[...] + p_ref[...]


def _merge_body(sc_ref, tc_ref, o_ref):
    o_ref[...] = sc_ref[...]


def kernel(x, position_embeddings):
    B, S, H = x.shape
    pf = position_embeddings[:S]

    sc_run = functools.partial(
        pl.kernel,
        mesh=plsc.VectorSubcoreMesh(core_axis_name="c", subcore_axis_name="s"),
        out_type=jax.ShapeDtypeStruct((B, _S1, H), x.dtype),
        scratch_types=(
            [pltpu.VMEM((_R, _H), jnp.float32) for _ in range(10)]
            + [pltpu.SemaphoreType.DMA for _ in range(4)]
        ),
    )(_sc_body)
    sc_out = sc_run(x, pf)

    blk0 = _S1 // _TC_BS
    nblk = (S - _S1) // _TC_BS
    tc_out = pl.pallas_call(
        _tc_body,
        grid=(nblk, B),
        in_specs=[
            pl.BlockSpec((1, _TC_BS, H), lambda i, j: (j, i + blk0, 0)),
            pl.BlockSpec((_TC_BS, H), lambda i, j: (i + blk0, 0)),
        ],
        out_specs=pl.BlockSpec((1, _TC_BS, H), lambda i, j: (j, i + blk0, 0)),
        out_shape=jax.ShapeDtypeStruct((B, S, H), x.dtype),
    )(x, pf)

    return pl.pallas_call(
        _merge_body,
        grid=(blk0, B),
        in_specs=[
            pl.BlockSpec((1, _TC_BS, H), lambda i, j: (j, i, 0)),
            pl.BlockSpec(memory_space=pl.ANY),
        ],
        out_specs=pl.BlockSpec((1, _TC_BS, H), lambda i, j: (j, i, 0)),
        out_shape=jax.ShapeDtypeStruct((B, S, H), x.dtype),
        input_output_aliases={1: 0},
    )(sc_out, tc_out)


# speedup vs baseline: 1.1835x; 1.1835x over previous
import functools
import jax
import jax.numpy as jnp
from jax import lax
from jax.experimental import pallas as pl
from jax.experimental.pallas import tpu as pltpu, tpu_sc as plsc

_B, _S, _H = 4, 4096, 1024
_S1 = 2048
_NC, _NS = 2, 16
_NW = _NC * _NS
_ROWS_W = _S1 // _NW
_R = 8
_NCH = _ROWS_W // _R
_TC_BS = 1024


def _sc_body(x_hbm, pos_hbm, out_hbm,
             pos0, pos1, xa0, xb0, xc0, xd0, xa1, xb1, xc1, xd1,
             sin0, sin1, sout0, sout1):
    wid = lax.axis_index("c") * _NS + lax.axis_index("s")
    posv = (pos0, pos1)
    xv = ((xa0, xb0, xc0, xd0), (xa1, xb1, xc1, xd1))
    sin = (sin0, sin1)
    sout = (sout0, sout1)

    def rbase(g):
        return wid * _ROWS_W + g * _R

    def issue_in(g, p):
        pltpu.async_copy(pos_hbm.at[pl.ds(rbase(g), _R), :], posv[p], sin[p])
        for b in range(_B):
            pltpu.async_copy(x_hbm.at[b, pl.ds(rbase(g), _R), :], xv[p][b], sin[p])

    def drain_in(p):
        pltpu.make_async_copy(pos_hbm.at[pl.ds(0, _R), :], posv[p], sin[p]).wait()
        for b in range(_B):
            pltpu.make_async_copy(x_hbm.at[0, pl.ds(0, _R), :], xv[p][b], sin[p]).wait()

    def issue_out(g, p):
        for b in range(_B):
            pltpu.async_copy(xv[p][b], out_hbm.at[b, pl.ds(rbase(g), _R), :], sout[p])

    def drain_out(p):
        for b in range(_B):
            pltpu.make_async_copy(xv[p][b], out_hbm.at[0, pl.ds(0, _R), :], sout[p]).wait()

    def compute(p):
        bufs = xv[p]
        pv_ref = posv[p]

        def row_body(r, acc):
            def col_body(j, acc2):
                cs = j * 16
                pv = pv_ref[r, pl.ds(cs, 16)]
                for b in range(_B):
                    plsc.addupdate(bufs[b].at[r, pl.ds(cs, 16)], pv)
                return acc2

            return lax.fori_loop(0, _H // 16, col_body, acc, unroll=8)

        lax.fori_loop(0, _R, row_body, 0)

    issue_in(0, 0)
    for g in range(_NCH):
        p = g % 2
        if g + 1 < _NCH:
            if g >= 1:
                drain_out(1 - p)
            issue_in(g + 1, 1 - p)
        drain_in(p)
        compute(p)
        issue_out(g, p)
    drain_out(0)
    drain_out(1)


def _tc_body(x_ref, p_ref, sc_ref, o_ref):
    o_ref[...] = x_ref[...] + p_ref[...]


def kernel(x, position_embeddings):
    B, S, H = x.shape
    pf = position_embeddings[:S]

    sc_run = functools.partial(
        pl.kernel,
        mesh=plsc.VectorSubcoreMesh(core_axis_name="c", subcore_axis_name="s"),
        out_type=jax.ShapeDtypeStruct((B, S, H), x.dtype),
        scratch_types=(
            [pltpu.VMEM((_R, _H), jnp.float32) for _ in range(10)]
            + [pltpu.SemaphoreType.DMA for _ in range(4)]
        ),
    )(_sc_body)
    sc_out = sc_run(x, pf)

    blk0 = _S1 // _TC_BS
    nblk = (S - _S1) // _TC_BS
    return pl.pallas_call(
        _tc_body,
        grid=(nblk, B),
        in_specs=[
            pl.BlockSpec((1, _TC_BS, H), lambda i, j: (j, i + blk0, 0)),
            pl.BlockSpec((_TC_BS, H), lambda i, j: (i + blk0, 0)),
            pl.BlockSpec(memory_space=pl.ANY),
        ],
        out_specs=pl.BlockSpec((1, _TC_BS, H), lambda i, j: (j, i + blk0, 0)),
        out_shape=jax.ShapeDtypeStruct((B, S, H), x.dtype),
        input_output_aliases={2: 0},
    )(x, pf, sc_out)


# speedup vs baseline: 1.2156x; 1.0271x over previous
import functools
import jax
import jax.numpy as jnp
from jax import lax
from jax.experimental import pallas as pl
from jax.experimental.pallas import tpu as pltpu, tpu_sc as plsc

_B, _S, _H = 4, 4096, 1024
_S1 = 2048
_NC, _NS = 2, 16
_NW = _NC * _NS
_ROWS_W = _S1 // _NW
_R = 8
_NCH = _ROWS_W // _R
_TC_BS = 2048


def _sc_body(x_hbm, pos_hbm, out_hbm,
             pos0, pos1, xa0, xb0, xc0, xd0, xa1, xb1, xc1, xd1,
             sin0, sin1, sout0, sout1):
    wid = lax.axis_index("c") * _NS + lax.axis_index("s")
    posv = (pos0, pos1)
    xv = ((xa0, xb0, xc0, xd0), (xa1, xb1, xc1, xd1))
    sin = (sin0, sin1)
    sout = (sout0, sout1)

    def rbase(g):
        return wid * _ROWS_W + g * _R

    def issue_in(g, p):
        pltpu.async_copy(pos_hbm.at[pl.ds(rbase(g), _R), :], posv[p], sin[p])
        for b in range(_B):
            pltpu.async_copy(x_hbm.at[b, pl.ds(rbase(g), _R), :], xv[p][b], sin[p])

    def drain_in(p):
        pltpu.make_async_copy(pos_hbm.at[pl.ds(0, _R), :], posv[p], sin[p]).wait()
        for b in range(_B):
            pltpu.make_async_copy(x_hbm.at[0, pl.ds(0, _R), :], xv[p][b], sin[p]).wait()

    def issue_out(g, p):
        for b in range(_B):
            pltpu.async_copy(xv[p][b], out_hbm.at[b, pl.ds(rbase(g), _R), :], sout[p])

    def drain_out(p):
        for b in range(_B):
            pltpu.make_async_copy(xv[p][b], out_hbm.at[0, pl.ds(0, _R), :], sout[p]).wait()

    def compute(p):
        bufs = xv[p]
        pv_ref = posv[p]

        def row_body(r, acc):
            def col_body(j, acc2):
                cs = j * 16
                pv = pv_ref[r, pl.ds(cs, 16)]
                for b in range(_B):
                    plsc.addupdate(bufs[b].at[r, pl.ds(cs, 16)], pv)
                return acc2

            return lax.fori_loop(0, _H // 16, col_body, acc, unroll=8)

        lax.fori_loop(0, _R, row_body, 0)

    issue_in(0, 0)
    for g in range(_NCH):
        p = g % 2
        if g + 1 < _NCH:
            if g >= 1:
                drain_out(1 - p)
            issue_in(g + 1, 1 - p)
        drain_in(p)
        compute(p)
        issue_out(g, p)
    drain_out(0)
    drain_out(1)


def _tc_body(x_ref, p_ref, sc_ref, o_ref):
    o_ref[...] = x_ref[...] + p_ref[...]


def kernel(x, position_embeddings):
    B, S, H = x.shape
    pf = position_embeddings[:S]

    sc_run = functools.partial(
        pl.kernel,
        mesh=plsc.VectorSubcoreMesh(core_axis_name="c", subcore_axis_name="s"),
        out_type=jax.ShapeDtypeStruct((B, S, H), x.dtype),
        scratch_types=(
            [pltpu.VMEM((_R, _H), jnp.float32) for _ in range(10)]
            + [pltpu.SemaphoreType.DMA for _ in range(4)]
        ),
    )(_sc_body)
    sc_out = sc_run(x, pf)

    blk0 = _S1 // _TC_BS
    nblk = (S - _S1) // _TC_BS
    return pl.pallas_call(
        _tc_body,
        grid=(nblk, B),
        in_specs=[
            pl.BlockSpec((1, _TC_BS, H), lambda i, j: (j, i + blk0, 0)),
            pl.BlockSpec((_TC_BS, H), lambda i, j: (i + blk0, 0)),
            pl.BlockSpec(memory_space=pl.ANY),
        ],
        out_specs=pl.BlockSpec((1, _TC_BS, H), lambda i, j: (j, i + blk0, 0)),
        out_shape=jax.ShapeDtypeStruct((B, S, H), x.dtype),
        input_output_aliases={2: 0},
    )(x, pf, sc_out)
